# baseline (speedup 1.0000x reference)
import functools

import jax
import jax.numpy as jnp
from jax import lax
from jax.experimental import pallas as pl
from jax.experimental.pallas import tpu as pltpu
from jax.experimental.pallas import tpu_sc as plsc

NUM_CORES = 2
NUM_SUBCORES = 16
NW = NUM_CORES * NUM_SUBCORES
CHUNK = 128


def kernel(labels, embedding_table):
    (B,) = labels.shape
    V, D = embedding_table.shape
    b_per_w = B // NW
    n_ch = b_per_w // CHUNK

    labels_3d = labels.astype(jnp.int32).reshape(NW, n_ch, CHUNK)
    mesh = plsc.VectorSubcoreMesh(core_axis_name="c", subcore_axis_name="s")

    @functools.partial(
        pl.kernel,
        mesh=mesh,
        out_type=jax.ShapeDtypeStruct((B, D), jnp.float32),
        scratch_types=[
            pltpu.VMEM((n_ch, CHUNK), jnp.int32),
            pltpu.VMEM((b_per_w, D), jnp.float32),
        ]
        + [pltpu.SemaphoreType.DMA] * n_ch
        + [pltpu.SemaphoreType.DMA],
    )
    def emb(table_hbm, labels_hbm, out_hbm, idx_v, rows_v, *sems):
        gsems, osem = sems[:n_ch], sems[n_ch]
        wid = lax.axis_index("s") * NUM_CORES + lax.axis_index("c")
        base = wid * b_per_w
        pltpu.sync_copy(labels_hbm.at[wid], idx_v)
        gathers = [None] * n_ch
        gathers[0] = pltpu.async_copy(
            table_hbm.at[idx_v.at[0]], rows_v.at[pl.ds(0, CHUNK)], gsems[0]
        )
        stores = []
        for j in range(n_ch):
            gathers[j].wait()
            if j + 1 < n_ch:
                gathers[j + 1] = pltpu.async_copy(
                    table_hbm.at[idx_v.at[j + 1]],
                    rows_v.at[pl.ds((j + 1) * CHUNK, CHUNK)],
                    gsems[j + 1],
                )
            stores.append(
                pltpu.async_copy(
                    rows_v.at[pl.ds(j * CHUNK, CHUNK)],
                    out_hbm.at[pl.ds(base + j * CHUNK, CHUNK)],
                    osem,
                )
            )
        for c in stores:
            c.wait()

    return emb(embedding_table, labels_3d)


# speedup vs baseline: 1.0019x; 1.0019x over previous
import functools

import jax
import jax.numpy as jnp
from jax import lax
from jax.experimental import pallas as pl
from jax.experimental.pallas import tpu as pltpu
from jax.experimental.pallas import tpu_sc as plsc

NUM_CORES = 2
NUM_SUBCORES = 16
B_CORE0 = 544
B_CORE1 = 480
B_PAIR = B_CORE0 + B_CORE1


def kernel(labels, embedding_table):
    (B,) = labels.shape
    V, D = embedding_table.shape

    labels_1d = labels.astype(jnp.int32)
    mesh = plsc.VectorSubcoreMesh(core_axis_name="c", subcore_axis_name="s")

    @functools.partial(
        pl.kernel,
        mesh=mesh,
        out_type=jax.ShapeDtypeStruct((B, D), jnp.float32),
        scratch_types=[
            pltpu.VMEM((B_CORE0,), jnp.int32),
            pltpu.VMEM((B_CORE0, D), jnp.float32),
            pltpu.SemaphoreType.DMA,
        ],
    )
    def emb(table_hbm, labels_hbm, out_hbm, idx_v, rows_v, sem):
        cid = lax.axis_index("c")
        sid = lax.axis_index("s")

        @pl.when(cid == 0)
        def _():
            base = sid * B_PAIR
            pltpu.sync_copy(
                labels_hbm.at[pl.ds(base, B_CORE0)], idx_v.at[pl.ds(0, B_CORE0)]
            )
            pltpu.async_copy(
                table_hbm.at[idx_v.at[pl.ds(0, B_CORE0)]],
                rows_v.at[pl.ds(0, B_CORE0)],
                sem,
            ).wait()
            pltpu.sync_copy(
                rows_v.at[pl.ds(0, B_CORE0)], out_hbm.at[pl.ds(base, B_CORE0)]
            )

        @pl.when(cid == 1)
        def _():
            base = sid * B_PAIR + B_CORE0
            pltpu.sync_copy(
                labels_hbm.at[pl.ds(base, B_CORE1)], idx_v.at[pl.ds(0, B_CORE1)]
            )
            pltpu.async_copy(
                table_hbm.at[idx_v.at[pl.ds(0, B_CORE1)]],
                rows_v.at[pl.ds(0, B_CORE1)],
                sem,
            ).wait()
            pltpu.sync_copy(
                rows_v.at[pl.ds(0, B_CORE1)], out_hbm.at[pl.ds(base, B_CORE1)]
            )

    return emb(embedding_table, labels_1d)
